# baseline (speedup 1.0000x reference)
import jax
import jax.numpy as jnp
from jax import lax
from jax.experimental import pallas as pl
from jax.experimental.pallas import tpu as pltpu
from jax.experimental.pallas import tpu_sc as plsc

B = 16
SEG = 2
L = 512
M = 510
OUT = M + 3
LANES = 16
NCHUNK = 33
OUT_PAD = NCHUNK * LANES
CLS_ID = 2
SEP_ID = 3
NC = 2
NS = 16


def _sc_body(tok_h, len_h, tokid_h, typ_h,
             tok_v, len_v, tid_v, typ_v, sem0, sem1, sem2):
  wid = lax.axis_index("s") * NC + lax.axis_index("c")
  b = wid % B
  task = wid // B
  iota = lax.iota(jnp.int32, LANES)
  bvec = jnp.broadcast_to(b, (LANES,)).astype(jnp.int32)

  @pl.when(task == 0)
  def _tokens():
    cp_tok0 = pltpu.async_copy(tok_h.at[b, 0], tok_v.at[pl.ds(0, L)], sem0)
    cp_tok1 = pltpu.async_copy(tok_h.at[b, 1], tok_v.at[pl.ds(L, L)], sem2)
    cp_len = pltpu.async_copy(len_h, len_v, sem1)
    cp_len.wait()
    zeros = jnp.zeros((LANES,), jnp.int32)
    l1 = plsc.load_gather(len_v, [bvec, zeros])
    l2 = plsc.load_gather(len_v, [bvec, zeros + 1])
    k1 = jnp.minimum(l1, jnp.maximum((M + 1) // 2, M - l2))
    k2 = jnp.minimum(l2, jnp.maximum(M // 2, M - l1))
    cp_tok0.wait()
    cp_tok1.wait()

    def chunk(c, carry):
      pos = iota + c * LANES
      idx1 = jnp.clip(pos - 1, 0, L - 1)
      idx2 = jnp.clip(pos - k1 - 2, 0, L - 1)
      t1 = plsc.load_gather(tok_v, [idx1])
      t2 = plsc.load_gather(tok_v, [idx2 + L])
      val = jnp.where(
          pos <= k1,
          jnp.where(pos == 0, CLS_ID, t1),
          jnp.where(
              pos == k1 + 1, SEP_ID,
              jnp.where(pos <= k1 + k2 + 1, t2,
                        jnp.where(pos == k1 + k2 + 2, SEP_ID, 0))))
      typ = jnp.where((pos > k1 + 1) & (pos <= k1 + k2 + 2), 1, 0)
      sl = pl.ds(c * LANES, LANES)
      tid_v[sl] = val.astype(jnp.int32)
      typ_v[sl] = typ.astype(jnp.int32)
      return carry

    lax.fori_loop(0, NCHUNK, chunk, 0, unroll=False)
    cp_o0 = pltpu.async_copy(tid_v, tokid_h.at[b], sem0)
    cp_o1 = pltpu.async_copy(typ_v, typ_h.at[b], sem1)
    cp_o0.wait()
    cp_o1.wait()


def _tc_mask_body(len_ref, st_ref, en_ref, sto_ref, eno_ref):
  l1 = len_ref[...][:, 0:1]
  col = lax.broadcasted_iota(jnp.int32, (B, L), 1)
  m = col < l1
  sto_ref[...] = jnp.where(m, st_ref[...], 0.0)
  eno_ref[...] = jnp.where(m, en_ref[...], 0.0)


@jax.jit
def kernel(tokens, lengths, starts, ends):
  mesh = plsc.VectorSubcoreMesh(
      core_axis_name="c", subcore_axis_name="s",
      num_cores=NC, num_subcores=NS)
  run_sc = pl.kernel(
      _sc_body,
      out_type=(
          jax.ShapeDtypeStruct((B, OUT_PAD), jnp.int32),
          jax.ShapeDtypeStruct((B, OUT_PAD), jnp.int32),
      ),
      mesh=mesh,
      compiler_params=pltpu.CompilerParams(needs_layout_passes=False),
      scratch_types=[
          pltpu.VMEM((SEG * L,), jnp.int32),
          pltpu.VMEM((B, SEG), jnp.int32),
          pltpu.VMEM((OUT_PAD,), jnp.int32),
          pltpu.VMEM((OUT_PAD,), jnp.int32),
          pltpu.SemaphoreType.DMA,
          pltpu.SemaphoreType.DMA,
          pltpu.SemaphoreType.DMA,
      ],
  )
  tokid, typ = run_sc(tokens, lengths)
  sto, eno = pl.pallas_call(
      _tc_mask_body,
      out_shape=(
          jax.ShapeDtypeStruct((B, L), jnp.float32),
          jax.ShapeDtypeStruct((B, L), jnp.float32),
      ),
  )(lengths, starts, ends)
  return tokid[:, :OUT], typ[:, :OUT], sto, eno


# speedup vs baseline: 1.0650x; 1.0650x over previous
import jax
import jax.numpy as jnp
from jax import lax
from jax.experimental import pallas as pl
from jax.experimental.pallas import tpu as pltpu
from jax.experimental.pallas import tpu_sc as plsc

B = 16
SEG = 2
L = 512
M = 510
OUT = M + 3
LANES = 16
NCHUNK = 33
OUT_PAD = NCHUNK * LANES
CLS_ID = 2
SEP_ID = 3
NC = 1
NS = 16


def _sc_body(tok_h, len_h, tokid_h, typ_h,
             tok_v, len_v, tid_v, typ_v, sem0, sem1, sem2):
  wid = lax.axis_index("s") * NC + lax.axis_index("c")
  b = wid % B
  task = wid // B
  iota = lax.iota(jnp.int32, LANES)
  bvec = jnp.broadcast_to(b, (LANES,)).astype(jnp.int32)

  @pl.when(task == 0)
  def _tokens():
    cp_tok0 = pltpu.async_copy(tok_h.at[b, 0], tok_v.at[pl.ds(0, L)], sem0)
    cp_tok1 = pltpu.async_copy(tok_h.at[b, 1], tok_v.at[pl.ds(L, L)], sem2)
    cp_len = pltpu.async_copy(len_h, len_v, sem1)
    cp_len.wait()
    l1 = plsc.load_gather(len_v, [bvec * 2])
    l2 = plsc.load_gather(len_v, [bvec * 2 + 1])
    k1 = jnp.minimum(l1, jnp.maximum((M + 1) // 2, M - l2))
    k2 = jnp.minimum(l2, jnp.maximum(M // 2, M - l1))
    cp_tok0.wait()
    cp_tok1.wait()

    def chunk(c, carry):
      pos = iota + c * LANES
      idx1 = jnp.clip(pos - 1, 0, L - 1)
      idx2 = jnp.clip(pos - k1 - 2, 0, L - 1)
      t1 = plsc.load_gather(tok_v, [idx1])
      t2 = plsc.load_gather(tok_v, [idx2 + L])
      val = jnp.where(
          pos <= k1,
          jnp.where(pos == 0, CLS_ID, t1),
          jnp.where(
              pos == k1 + 1, SEP_ID,
              jnp.where(pos <= k1 + k2 + 1, t2,
                        jnp.where(pos == k1 + k2 + 2, SEP_ID, 0))))
      typ = jnp.where((pos > k1 + 1) & (pos <= k1 + k2 + 2), 1, 0)
      sl = pl.ds(c * LANES, LANES)
      tid_v[sl] = val.astype(jnp.int32)
      typ_v[sl] = typ.astype(jnp.int32)
      return carry

    lax.fori_loop(0, NCHUNK, chunk, 0, unroll=False)
    cp_o0 = pltpu.async_copy(tid_v, tokid_h.at[b], sem0)
    cp_o1 = pltpu.async_copy(typ_v, typ_h.at[b], sem1)
    cp_o0.wait()
    cp_o1.wait()


def _tc_mask_body(len_ref, st_ref, en_ref, sto_ref, eno_ref):
  l1 = len_ref[...][:, 0:1]
  col = lax.broadcasted_iota(jnp.int32, (B, L), 1)
  m = col < l1
  sto_ref[...] = jnp.where(m, st_ref[...], 0.0)
  eno_ref[...] = jnp.where(m, en_ref[...], 0.0)


@jax.jit
def kernel(tokens, lengths, starts, ends):
  mesh = plsc.VectorSubcoreMesh(
      core_axis_name="c", subcore_axis_name="s",
      num_cores=NC, num_subcores=NS)
  run_sc = pl.kernel(
      _sc_body,
      out_type=(
          jax.ShapeDtypeStruct((B, OUT_PAD), jnp.int32),
          jax.ShapeDtypeStruct((B, OUT_PAD), jnp.int32),
      ),
      mesh=mesh,
      compiler_params=pltpu.CompilerParams(needs_layout_passes=False),
      scratch_types=[
          pltpu.VMEM((SEG * L,), jnp.int32),
          pltpu.VMEM((B * SEG,), jnp.int32),
          pltpu.VMEM((OUT_PAD,), jnp.int32),
          pltpu.VMEM((OUT_PAD,), jnp.int32),
          pltpu.SemaphoreType.DMA,
          pltpu.SemaphoreType.DMA,
          pltpu.SemaphoreType.DMA,
      ],
  )
  tokid, typ = run_sc(tokens, lengths.reshape(B * SEG))
  sto, eno = pl.pallas_call(
      _tc_mask_body,
      out_shape=(
          jax.ShapeDtypeStruct((B, L), jnp.float32),
          jax.ShapeDtypeStruct((B, L), jnp.float32),
      ),
  )(lengths, starts, ends)
  return tokid[:, :OUT], typ[:, :OUT], sto, eno


# speedup vs baseline: 1.0714x; 1.0060x over previous
import jax
import jax.numpy as jnp
from jax import lax
from jax.experimental import pallas as pl
from jax.experimental.pallas import tpu as pltpu
from jax.experimental.pallas import tpu_sc as plsc

B = 16
SEG = 2
L = 512
M = 510
OUT = M + 3
LANES = 16
NCHUNK = 33
OUT_PAD = NCHUNK * LANES
CLS_ID = 2
SEP_ID = 3
NC = 1
NS = 16


def _sc_body(tok_h, len_h, tokid_h, typ_h,
             tok_v, len_v, tid_v, typ_v, sem0, sem1, sem2):
  b = lax.axis_index("s") * NC + lax.axis_index("c")
  iota = lax.iota(jnp.int32, LANES)
  bvec = jnp.broadcast_to(b, (LANES,)).astype(jnp.int32)

  cp_tok0 = pltpu.async_copy(tok_h.at[b, 0], tok_v.at[pl.ds(0, L)], sem0)
  cp_tok1 = pltpu.async_copy(tok_h.at[b, 1], tok_v.at[pl.ds(L, L)], sem2)
  cp_len = pltpu.async_copy(len_h, len_v, sem1)
  cp_len.wait()
  l1 = plsc.load_gather(len_v, [bvec * 2])
  l2 = plsc.load_gather(len_v, [bvec * 2 + 1])
  k1 = jnp.minimum(l1, jnp.maximum((M + 1) // 2, M - l2))
  k2 = jnp.minimum(l2, jnp.maximum(M // 2, M - l1))
  cp_tok0.wait()
  cp_tok1.wait()

  def chunk(c, carry):
    pos = iota + c * LANES
    idx1 = jnp.clip(pos - 1, 0, L - 1)
    idx2 = jnp.clip(pos - k1 - 2, 0, L - 1)
    t1 = plsc.load_gather(tok_v, [idx1])
    t2 = plsc.load_gather(tok_v, [idx2 + L])
    val = jnp.where(
        pos <= k1,
        jnp.where(pos == 0, CLS_ID, t1),
        jnp.where(
            pos == k1 + 1, SEP_ID,
            jnp.where(pos <= k1 + k2 + 1, t2,
                      jnp.where(pos == k1 + k2 + 2, SEP_ID, 0))))
    typ = jnp.where((pos > k1 + 1) & (pos <= k1 + k2 + 2), 1, 0)
    sl = pl.ds(c * LANES, LANES)
    tid_v[sl] = val.astype(jnp.int32)
    typ_v[sl] = typ.astype(jnp.int32)
    return carry

  lax.fori_loop(0, NCHUNK, chunk, 0, unroll=False)
  cp_o0 = pltpu.async_copy(tid_v, tokid_h.at[b], sem0)
  cp_o1 = pltpu.async_copy(typ_v, typ_h.at[b], sem1)
  cp_o0.wait()
  cp_o1.wait()


def _tc_mask_body(len_ref, st_ref, en_ref, sto_ref, eno_ref):
  l1 = len_ref[...][:, 0:1]
  col = lax.broadcasted_iota(jnp.int32, (B, L), 1)
  m = col < l1
  sto_ref[...] = jnp.where(m, st_ref[...], 0.0)
  eno_ref[...] = jnp.where(m, en_ref[...], 0.0)


@jax.jit
def kernel(tokens, lengths, starts, ends):
  mesh = plsc.VectorSubcoreMesh(
      core_axis_name="c", subcore_axis_name="s",
      num_cores=NC, num_subcores=NS)
  run_sc = pl.kernel(
      _sc_body,
      out_type=(
          jax.ShapeDtypeStruct((B, OUT_PAD), jnp.int32),
          jax.ShapeDtypeStruct((B, OUT_PAD), jnp.int32),
      ),
      mesh=mesh,
      compiler_params=pltpu.CompilerParams(needs_layout_passes=False),
      scratch_types=[
          pltpu.VMEM((SEG * L,), jnp.int32),
          pltpu.VMEM((B * SEG,), jnp.int32),
          pltpu.VMEM((OUT_PAD,), jnp.int32),
          pltpu.VMEM((OUT_PAD,), jnp.int32),
          pltpu.SemaphoreType.DMA,
          pltpu.SemaphoreType.DMA,
          pltpu.SemaphoreType.DMA,
      ],
  )
  tokid, typ = run_sc(tokens, lengths.reshape(B * SEG))
  sto, eno = pl.pallas_call(
      _tc_mask_body,
      out_shape=(
          jax.ShapeDtypeStruct((B, L), jnp.float32),
          jax.ShapeDtypeStruct((B, L), jnp.float32),
      ),
  )(lengths, starts, ends)
  return tokid[:, :OUT], typ[:, :OUT], sto, eno
